# baseline (speedup 1.0000x reference)
import functools

import jax
import jax.numpy as jnp
from jax import lax
from jax.experimental import pallas as pl
from jax.experimental.pallas import tpu as pltpu
from jax.experimental.pallas import tpu_sc as plsc

CELLS = 16384
OUT_DIM = 32
FEATS = 64

NUM_SC = 2
NUM_TILES = 16
NUM_WORKERS = NUM_SC * NUM_TILES
CPT = CELLS // NUM_WORKERS
CHUNK = 128
NCHUNK = CPT // CHUNK
ROWS_PER_TILE = CELLS // NUM_TILES

BLOCK = 256
GRID = CELLS // BLOCK


def _sc_body(idx_hbm, mask_hbm, zero_hbm, out_hbm,
             idx0, idx1, idx2, idx3, mask_v, acc, sem):
    c = lax.axis_index("c")
    s = lax.axis_index("s")
    wid = s * NUM_SC + c

    zrow0 = s * ROWS_PER_TILE
    pltpu.sync_copy(zero_hbm.at[pl.ds(zrow0, ROWS_PER_TILE)],
                    acc.at[pl.ds(zrow0, ROWS_PER_TILE)])
    plsc.subcore_barrier()

    base = wid * CPT
    idx_refs = (idx0, idx1, idx2, idx3)
    fetches = []
    for j in range(NCHUNK):
        fetches.append(
            pltpu.async_copy(idx_hbm.at[pl.ds(base + j * CHUNK, CHUNK)],
                             idx_refs[j], sem))
        fetches.append(
            pltpu.async_copy(mask_hbm.at[pl.ds(base + j * CHUNK, CHUNK)],
                             mask_v.at[j], sem))
    for f in fetches:
        f.wait()

    for j in range(NCHUNK):
        pltpu.sync_copy(mask_v.at[j], acc.at[idx_refs[j]], add=True)

    plsc.subcore_barrier()
    pltpu.sync_copy(acc.at[pl.ds(zrow0, ROWS_PER_TILE)],
                    out_hbm.at[pl.ds(c * CELLS + zrow0, ROWS_PER_TILE)])


@functools.cache
def _sc_scatter():
    mesh = plsc.VectorSubcoreMesh(core_axis_name="c", subcore_axis_name="s")
    return pl.kernel(
        _sc_body,
        out_type=jax.ShapeDtypeStruct((NUM_SC * CELLS, OUT_DIM), jnp.float32),
        mesh=mesh,
        scratch_types=[
            pltpu.VMEM((CHUNK,), jnp.int32),
            pltpu.VMEM((CHUNK,), jnp.int32),
            pltpu.VMEM((CHUNK,), jnp.int32),
            pltpu.VMEM((CHUNK,), jnp.int32),
            pltpu.VMEM((NCHUNK, CHUNK, OUT_DIM), jnp.float32),
            pltpu.VMEM_SHARED((CELLS, OUT_DIM), jnp.float32),
            pltpu.SemaphoreType.DMA,
        ],
        compiler_params=pltpu.CompilerParams(use_tc_tiling_on_sc=False),
    )


def _tc_body(x_ref, w_ref, s_ref, out_ref):
    y = jnp.dot(x_ref[...], w_ref[...], preferred_element_type=jnp.float32)
    out_ref[...] = y + s_ref[0] + s_ref[1]


@jax.jit
def kernel(x, index, mask, w1, w2):
    w = jnp.concatenate([w1, w2], axis=-1)
    w_blk = jnp.kron(jnp.eye(OUT_DIM, dtype=jnp.float32), w.reshape(FEATS, 1))
    w_blk = w_blk.astype(jnp.bfloat16)
    x2 = x.reshape(CELLS, OUT_DIM * FEATS).astype(jnp.bfloat16)
    zero = jnp.zeros((CELLS, OUT_DIM), jnp.float32)
    s = _sc_scatter()(index.astype(jnp.int32), mask, zero)
    s = s.reshape(NUM_SC, CELLS, OUT_DIM)
    return pl.pallas_call(
        _tc_body,
        grid=(GRID,),
        in_specs=[
            pl.BlockSpec((BLOCK, OUT_DIM * FEATS), lambda i: (i, 0)),
            pl.BlockSpec((OUT_DIM * FEATS, OUT_DIM), lambda i: (0, 0)),
            pl.BlockSpec((NUM_SC, BLOCK, OUT_DIM), lambda i: (0, i, 0)),
        ],
        out_specs=pl.BlockSpec((BLOCK, OUT_DIM), lambda i: (i, 0)),
        out_shape=jax.ShapeDtypeStruct((CELLS, OUT_DIM), jnp.float32),
        compiler_params=pltpu.CompilerParams(
            dimension_semantics=("arbitrary",),
            allow_input_fusion=[True, False, False],
        ),
    )(x2, w_blk, s)


# speedup vs baseline: 1.1952x; 1.1952x over previous
import functools

import jax
import jax.numpy as jnp
from jax import lax
from jax.experimental import pallas as pl
from jax.experimental.pallas import tpu as pltpu
from jax.experimental.pallas import tpu_sc as plsc

CELLS = 16384
OUT_DIM = 32
FEATS = 64

NUM_SC = 2
NUM_TILES = 16
NUM_WORKERS = NUM_SC * NUM_TILES
CPT = CELLS // NUM_WORKERS
CHUNK = 128
NCHUNK = CPT // CHUNK
ROWS_PER_TILE = CELLS // NUM_TILES

BLOCK = 1024
GRID = CELLS // BLOCK


def _sc_body(idx_hbm, mask_hbm, zero_hbm, out_hbm,
             idx0, idx1, idx2, idx3, mask_v, acc, sem):
    c = lax.axis_index("c")
    s = lax.axis_index("s")
    wid = s * NUM_SC + c

    zrow0 = s * ROWS_PER_TILE
    pltpu.sync_copy(zero_hbm.at[pl.ds(zrow0, ROWS_PER_TILE)],
                    acc.at[pl.ds(zrow0, ROWS_PER_TILE)])
    plsc.subcore_barrier()

    base = wid * CPT
    idx_refs = (idx0, idx1, idx2, idx3)
    fetches = []
    for j in range(NCHUNK):
        fetches.append(
            pltpu.async_copy(idx_hbm.at[pl.ds(base + j * CHUNK, CHUNK)],
                             idx_refs[j], sem))
        fetches.append(
            pltpu.async_copy(mask_hbm.at[pl.ds(base + j * CHUNK, CHUNK)],
                             mask_v.at[j], sem))
    for f in fetches:
        f.wait()

    for j in range(NCHUNK):
        pltpu.sync_copy(mask_v.at[j], acc.at[idx_refs[j]], add=True)

    plsc.subcore_barrier()
    pltpu.sync_copy(acc.at[pl.ds(zrow0, ROWS_PER_TILE)],
                    out_hbm.at[pl.ds(c * CELLS + zrow0, ROWS_PER_TILE)])


@functools.cache
def _sc_scatter():
    mesh = plsc.VectorSubcoreMesh(core_axis_name="c", subcore_axis_name="s")
    return pl.kernel(
        _sc_body,
        out_type=jax.ShapeDtypeStruct((NUM_SC * CELLS, OUT_DIM), jnp.float32),
        mesh=mesh,
        scratch_types=[
            pltpu.VMEM((CHUNK,), jnp.int32),
            pltpu.VMEM((CHUNK,), jnp.int32),
            pltpu.VMEM((CHUNK,), jnp.int32),
            pltpu.VMEM((CHUNK,), jnp.int32),
            pltpu.VMEM((NCHUNK, CHUNK, OUT_DIM), jnp.float32),
            pltpu.VMEM_SHARED((CELLS, OUT_DIM), jnp.float32),
            pltpu.SemaphoreType.DMA,
        ],
        compiler_params=pltpu.CompilerParams(use_tc_tiling_on_sc=False),
    )


def _tc_body(x_ref, w_ref, s_ref, out_ref):
    y = jnp.dot(x_ref[...], w_ref[...], preferred_element_type=jnp.float32)
    out_ref[...] = y + s_ref[0] + s_ref[1]


@jax.jit
def kernel(x, index, mask, w1, w2):
    w = jnp.concatenate([w1, w2], axis=-1)
    w_blk = jnp.kron(jnp.eye(OUT_DIM, dtype=jnp.float32), w.reshape(FEATS, 1))
    x2 = x.reshape(CELLS, OUT_DIM * FEATS)
    zero = jnp.zeros((CELLS, OUT_DIM), jnp.float32)
    s = _sc_scatter()(index.astype(jnp.int32), mask, zero)
    s = s.reshape(NUM_SC, CELLS, OUT_DIM)
    return pl.pallas_call(
        _tc_body,
        grid=(GRID,),
        in_specs=[
            pl.BlockSpec((BLOCK, OUT_DIM * FEATS), lambda i: (i, 0)),
            pl.BlockSpec((OUT_DIM * FEATS, OUT_DIM), lambda i: (0, 0)),
            pl.BlockSpec((NUM_SC, BLOCK, OUT_DIM), lambda i: (0, i, 0)),
        ],
        out_specs=pl.BlockSpec((BLOCK, OUT_DIM), lambda i: (i, 0)),
        out_shape=jax.ShapeDtypeStruct((CELLS, OUT_DIM), jnp.float32),
        compiler_params=pltpu.CompilerParams(
            dimension_semantics=("arbitrary",),
        ),
    )(x2, w_blk, s)
